# baseline (speedup 1.0000x reference)
import functools

import jax
import jax.numpy as jnp
from jax import lax
from jax.experimental import pallas as pl
from jax.experimental.pallas import tpu as pltpu
from jax.experimental.pallas import tpu_sc as plsc

N_SITES = 200
D_MODEL = 64
BATCH = 4096

NC, NS, L = 2, 16, 16
NW = NC * NS
TOK_TOTAL = BATCH * N_SITES
TOK_PER_W = TOK_TOTAL // NW
ROWS_PER_CHUNK = 4
CHUNK = ROWS_PER_CHUNK * N_SITES
N_CHUNKS = TOK_PER_W // CHUNK
GATHER_W = 128
CHUNK_PAD = -(-CHUNK // GATHER_W) * GATHER_W
N_GATHERS = CHUNK_PAD // GATHER_W
NBUF = 2

_ROW_OFFS = tuple(range(0, N_SITES - L + 1, L)) + (N_SITES - L,)


def _sc_embed(n_flat, table):
    mesh = plsc.VectorSubcoreMesh(
        core_axis_name="c", subcore_axis_name="s", num_cores=NC, num_subcores=NS
    )

    scratch = []
    for _ in range(NBUF):
        scratch += [
            pltpu.VMEM((ROWS_PER_CHUNK, 2 * N_SITES), jnp.int32),
            pltpu.VMEM((CHUNK_PAD,), jnp.int32),
            pltpu.VMEM((CHUNK_PAD, D_MODEL), jnp.float32),
            pltpu.SemaphoreType.DMA,
            pltpu.SemaphoreType.DMA,
            pltpu.SemaphoreType.DMA,
        ]
    scratch.append(pltpu.VMEM_SHARED((4, D_MODEL), jnp.float32))

    @functools.partial(
        pl.kernel,
        out_type=jax.ShapeDtypeStruct((TOK_TOTAL, D_MODEL), jnp.float32),
        mesh=mesh,
        scratch_types=scratch,
        compiler_params=pltpu.CompilerParams(use_tc_tiling_on_sc=False),
    )
    def k(n_hbm, table_hbm, out_hbm, *s):
        bufs = [s[i * 6:(i + 1) * 6] for i in range(NBUF)]
        tab_v = s[NBUF * 6]

        wid = lax.axis_index("s") * NC + lax.axis_index("c")
        base = wid * TOK_PER_W
        row_base = wid * (TOK_PER_W // N_SITES)

        @pl.when(lax.axis_index("s") == 0)
        def _stage():
            pltpu.sync_copy(table_hbm, tab_v)

        plsc.subcore_barrier()

        zeros = jnp.zeros((L,), jnp.int32)
        for b in range(NBUF):
            idx_v = bufs[b][1]
            for g in range((CHUNK_PAD - CHUNK) // L):
                idx_v[pl.ds(CHUNK + g * L, L)] = zeros

        def fire_in(c, b):
            n_v, _, _, sem_i, _, _ = bufs[b]
            r0 = row_base + c * ROWS_PER_CHUNK
            pltpu.async_copy(n_hbm.at[pl.ds(r0, ROWS_PER_CHUNK)], n_v, sem_i)

        for b in range(NBUF):
            fire_in(b, b)

        def pair_body(cp, _):
            for b in range(NBUF):
                n_v, idx_v, rows_v, sem_i, sem_g, sem_o = bufs[b]
                c = cp * NBUF + b
                t0 = base + c * CHUNK

                pltpu.make_async_copy(
                    n_hbm.at[pl.ds(0, ROWS_PER_CHUNK)], n_v, sem_i
                ).wait()

                for r in range(ROWS_PER_CHUNK):
                    for o in _ROW_OFFS:
                        d = n_v[r, pl.ds(o, L)]
                        u = n_v[r, pl.ds(N_SITES + o, L)]
                        idx_v[pl.ds(r * N_SITES + o, L)] = u + d + d

                @pl.when(cp > 0)
                def _drain_prev():
                    pltpu.make_async_copy(
                        rows_v.at[pl.ds(0, CHUNK)],
                        out_hbm.at[pl.ds(0, CHUNK)],
                        sem_o,
                    ).wait()

                descs = []
                for j in range(N_GATHERS):
                    descs.append(
                        pltpu.async_copy(
                            tab_v.at[idx_v.at[pl.ds(j * GATHER_W, GATHER_W)]],
                            rows_v.at[pl.ds(j * GATHER_W, GATHER_W)],
                            sem_g,
                        )
                    )
                for desc in descs:
                    desc.wait()

                pltpu.async_copy(
                    rows_v.at[pl.ds(0, CHUNK)], out_hbm.at[pl.ds(t0, CHUNK)], sem_o
                )

                @pl.when(cp < N_CHUNKS // NBUF - 1)
                def _prefetch():
                    fire_in(c + NBUF, b)
            return 0

        lax.fori_loop(0, N_CHUNKS // NBUF, pair_body, 0)

        for b in range(NBUF):
            _, _, rows_v, _, _, sem_o = bufs[b]
            pltpu.make_async_copy(
                rows_v.at[pl.ds(0, CHUNK)], out_hbm.at[pl.ds(0, CHUNK)], sem_o
            ).wait()

    return k(n_flat, table)


def kernel(n_flat, embed_table):
    out = _sc_embed(jnp.asarray(n_flat), embed_table)
    return out.reshape(BATCH, N_SITES, D_MODEL)


# speedup vs baseline: 1.4139x; 1.4139x over previous
import functools

import jax
import jax.numpy as jnp
from jax import lax
from jax.experimental import pallas as pl
from jax.experimental.pallas import tpu as pltpu
from jax.experimental.pallas import tpu_sc as plsc

N_SITES = 200
D_MODEL = 64
BATCH = 4096

NC, NS, L = 2, 16, 16
NW = NC * NS
ROWS_PER_W = BATCH // NW
SUPER = 8
N_SUPER = ROWS_PER_W // SUPER
OUT_ROWS = 2
OUT_TOK = OUT_ROWS * N_SITES
N_OUT = SUPER // OUT_ROWS
N_GRP = (N_SITES + L - 1) // L

_ROW_OFFS = tuple(range(0, 161, 16)) + (168, 184)


def _sc_embed(n_flat, table_flat):
    mesh = plsc.VectorSubcoreMesh(
        core_axis_name="c", subcore_axis_name="s", num_cores=NC, num_subcores=NS
    )

    scratch = []
    for _ in range(2):
        scratch += [
            pltpu.VMEM((SUPER, 2 * N_SITES), jnp.int32),
            pltpu.VMEM((SUPER * N_SITES,), jnp.int32),
            pltpu.VMEM((OUT_ROWS, N_SITES, D_MODEL), jnp.float32),
            pltpu.SemaphoreType.DMA,
            pltpu.SemaphoreType.DMA,
        ]
    scratch.append(pltpu.VMEM((4 * D_MODEL,), jnp.float32))

    @functools.partial(
        pl.kernel,
        out_type=jax.ShapeDtypeStruct((BATCH, N_SITES, D_MODEL), jnp.float32),
        mesh=mesh,
        scratch_types=scratch,
        compiler_params=pltpu.CompilerParams(use_tc_tiling_on_sc=True),
    )
    def k(n_hbm, table_hbm, out_hbm, *s):
        bufs = [s[i * 5:(i + 1) * 5] for i in range(2)]
        tab_v = s[10]

        wid = lax.axis_index("s") * NC + lax.axis_index("c")
        row0 = wid * ROWS_PER_W

        pltpu.sync_copy(table_hbm, tab_v)
        tf = [
            [tab_v[pl.ds(r * D_MODEL + kk * L, L)] for kk in range(D_MODEL // L)]
            for r in range(4)
        ]
        xa = [tf[1][kk] - tf[0][kk] for kk in range(D_MODEL // L)]
        xb = [tf[2][kk] - tf[0][kk] for kk in range(D_MODEL // L)]
        xc = [
            tf[3][kk] - tf[2][kk] - tf[1][kk] + tf[0][kk]
            for kk in range(D_MODEL // L)
        ]
        t0 = tf[0]

        def fire_in(sc8, b):
            n_v, _, _, sem_i, _ = bufs[b]
            pltpu.async_copy(
                n_hbm.at[pl.ds(row0 + sc8 * SUPER, SUPER)], n_v, sem_i
            )

        for b in range(2):
            fire_in(b, b)

        def pair_body(p, _):
            for b in range(2):
                sc8 = p * 2 + b
                n_v, tok_v, _, sem_i, _ = bufs[b]
                pltpu.make_async_copy(
                    n_hbm.at[pl.ds(0, SUPER)], n_v, sem_i
                ).wait()

                for r in range(SUPER):
                    for o in _ROW_OFFS:
                        d = n_v[r, pl.ds(o, L)]
                        u = n_v[r, pl.ds(N_SITES + o, L)]
                        tok_v[pl.ds(r * N_SITES + o, L)] = u + d + d

                @pl.when(sc8 < N_SUPER - 2)
                def _prefetch():
                    fire_in(sc8 + 2, b)

                for oc in range(N_OUT):
                    ob = oc % 2
                    _, _, out_v, _, sem_o = bufs[ob]

                    if oc >= 2:
                        pltpu.make_async_copy(
                            out_v, out_hbm.at[pl.ds(0, OUT_ROWS)], sem_o
                        ).wait()
                    else:

                        @pl.when(sc8 > 0)
                        def _drain():
                            pltpu.make_async_copy(
                                out_v,
                                out_hbm.at[pl.ds(0, OUT_ROWS)],
                                sem_o,
                            ).wait()

                    for r2 in range(OUT_ROWS):

                        def grp_body(g, _, r2=r2, oc=oc, tok_v=tok_v,
                                     out_v=out_v):
                            o = jnp.minimum(g * L, N_SITES - L)
                            toks = tok_v[
                                pl.ds(oc * OUT_TOK + r2 * N_SITES + o, L)
                            ]
                            for i in range(L):
                                w = toks[i]
                                uf_s = (w & 1).astype(jnp.float32)
                                df_s = ((w >> 1) & 1).astype(jnp.float32)
                                uf = jnp.full((L,), uf_s, jnp.float32)
                                df = jnp.full((L,), df_s, jnp.float32)
                                udf = jnp.full((L,), uf_s * df_s, jnp.float32)
                                for kk in range(D_MODEL // L):
                                    out_v[r2, o + i, pl.ds(kk * L, L)] = (
                                        t0[kk]
                                        + uf * xa[kk]
                                        + df * xb[kk]
                                        + udf * xc[kk]
                                    )
                            return 0

                        lax.fori_loop(0, N_GRP, grp_body, 0)

                    pltpu.async_copy(
                        out_v,
                        out_hbm.at[
                            pl.ds(row0 + sc8 * SUPER + oc * OUT_ROWS, OUT_ROWS)
                        ],
                        sem_o,
                    )

            return 0

        lax.fori_loop(0, N_SUPER // 2, pair_body, 0)

        for b in range(2):
            _, _, out_v, _, sem_o = bufs[b]
            pltpu.make_async_copy(
                out_v, out_hbm.at[pl.ds(0, OUT_ROWS)], sem_o
            ).wait()

    return k(n_flat, table_flat)


def kernel(n_flat, embed_table):
    return _sc_embed(
        jnp.asarray(n_flat), jnp.reshape(embed_table, (4 * D_MODEL,))
    )
